# baseline (speedup 1.0000x reference)
import functools

import jax
import jax.numpy as jnp
from jax import lax
from jax.experimental import pallas as pl
from jax.experimental.pallas import tpu as pltpu
from jax.experimental.pallas import tpu_sc as plsc

_B, _C, _H, _W = 32, 768, 32, 32
_HW = _H * _W
_P = 400
_K1 = max(int(0.3 * _HW), 1)
_LANES = 16
_NSLICES = _HW // _LANES

_DN_RHS_T = (((1,), (1,)), ((), ()))


_HH = _HW // 2


def _stage1_body(xa_ref, xb_ref, pa_ref, pb_ref, phi1_ref, phi3_ref,
                 p1_ref, gap3_ref):
    phi1 = phi1_ref[...]
    phi3 = phi3_ref[...]
    p1_ref[0, :_HH] = jnp.dot(xa_ref[0], phi1,
                              preferred_element_type=jnp.float32)
    p1_ref[0, _HH:] = jnp.dot(xb_ref[0], phi1,
                              preferred_element_type=jnp.float32)
    pa = pa_ref[0]
    pb = pb_ref[0]
    p3a = jnp.dot(pa, phi3, preferred_element_type=jnp.float32)
    p3b = jnp.dot(pb, phi3, preferred_element_type=jnp.float32)
    mn = jnp.minimum(jnp.min(p3a), jnp.min(p3b))
    ga = jnp.sum(pa * (p3a > mn).astype(jnp.float32), axis=0, keepdims=True)
    gb = jnp.sum(pb * (p3b > mn).astype(jnp.float32), axis=0, keepdims=True)
    gap3_ref[0] = (ga + gb) * (1.0 / _HW)


def _stage1(xt, proxyt, phi1_col, phi3_col):
    return pl.pallas_call(
        _stage1_body,
        grid=(_B,),
        in_specs=[
            pl.BlockSpec((1, _HH, _C), lambda b: (b, 0, 0)),
            pl.BlockSpec((1, _HH, _C), lambda b: (b, 1, 0)),
            pl.BlockSpec((1, _HH, _P), lambda b: (b, 0, 0)),
            pl.BlockSpec((1, _HH, _P), lambda b: (b, 1, 0)),
            pl.BlockSpec((_C, 1), lambda b: (0, 0)),
            pl.BlockSpec((_P, 1), lambda b: (0, 0)),
        ],
        out_specs=[
            pl.BlockSpec((1, _HW, 1), lambda b: (b, 0, 0)),
            pl.BlockSpec((1, 1, _P), lambda b: (b, 0, 0)),
        ],
        out_shape=[
            jax.ShapeDtypeStruct((_B, _HW, 1), jnp.float32),
            jax.ShapeDtypeStruct((_B, 1, _P), jnp.float32),
        ],
        compiler_params=pltpu.CompilerParams(
            dimension_semantics=("arbitrary",)),
    )(xt, xt, proxyt, proxyt, phi1_col, phi3_col)


def _sc_body(p_hbm, mask_hbm, row_v, u_v, m_v):
    wid = lax.axis_index("s") * 2 + lax.axis_index("c")
    pltpu.sync_copy(p_hbm.at[wid], row_v)

    for j in range(_NSLICES):
        f = row_v[pl.ds(j * _LANES, _LANES)]
        u = lax.bitcast_convert_type(f, jnp.uint32)
        u = jnp.where(u >= jnp.uint32(0x80000000), ~u, u | jnp.uint32(0x80000000))
        u_v[pl.ds(j * _LANES, _LANES)] = u

    def _bit_step(s, t):
        bit = jnp.uint32(1) << (jnp.uint32(31) - s.astype(jnp.uint32))
        cand = t | bit
        acc = jnp.zeros((_LANES,), jnp.int32)
        for j in range(_NSLICES):
            u = u_v[pl.ds(j * _LANES, _LANES)]
            acc = acc + jnp.where(u >= cand, jnp.int32(1), jnp.int32(0))
        total = jnp.sum(acc)
        return jnp.where(total >= jnp.int32(_K1), cand, t)

    t = lax.fori_loop(0, 32, _bit_step, jnp.uint32(0))

    for j in range(_NSLICES):
        u = u_v[pl.ds(j * _LANES, _LANES)]
        m_v[pl.ds(j * _LANES, _LANES)] = jnp.where(u > t, 1.0, 0.0).astype(
            jnp.float32)
    pltpu.sync_copy(m_v, mask_hbm.at[wid])


@functools.cache
def _sc_topk_mask():
    return functools.partial(
        pl.kernel,
        out_type=jax.ShapeDtypeStruct((_B, _HW), jnp.float32),
        mesh=plsc.VectorSubcoreMesh(core_axis_name="c", subcore_axis_name="s"),
        compiler_params=pltpu.CompilerParams(needs_layout_passes=False),
        scratch_types=[
            pltpu.VMEM((_HW,), jnp.float32),
            pltpu.VMEM((_HW,), jnp.uint32),
            pltpu.VMEM((_HW,), jnp.float32),
        ],
    )(_sc_body)


def _stage3_body(xa_ref, xb_ref, m1_ref, gap3_ref, w1_ref, b1_ref, w3_ref,
                 b3_ref, blk_ref, blkb_ref, c2_ref, out_ref):
    xa = xa_ref[0]
    xb = xb_ref[0]
    m1 = m1_ref[0]
    ga = jnp.sum(xa * m1[:_HH], axis=0, keepdims=True)
    gb = jnp.sum(xb * m1[_HH:], axis=0, keepdims=True)
    gap1 = (ga + gb) * (1.0 / _HW)
    cl1 = lax.dot_general(gap1, w1_ref[...], _DN_RHS_T,
                          preferred_element_type=jnp.float32) + b1_ref[...]
    cl3 = lax.dot_general(gap3_ref[0], w3_ref[...], _DN_RHS_T,
                          preferred_element_type=jnp.float32) + b3_ref[...]
    cat = jnp.concatenate([cl1, c2_ref[...], cl3], axis=1)
    apre = lax.dot_general(cat, blk_ref[...], _DN_RHS_T,
                           preferred_element_type=jnp.float32) + blkb_ref[...]
    a = 1.0 + jnp.tanh(apre)
    out_ref[0, :_HH] = xa * a
    out_ref[0, _HH:] = xb * a


def _stage3(xt, mask3d, gap3, w1, b1r, w3, b3r, blk, blkbr, c2r):
    const = lambda b: (0, 0)
    return pl.pallas_call(
        _stage3_body,
        grid=(_B,),
        in_specs=[
            pl.BlockSpec((1, _HH, _C), lambda b: (b, 0, 0)),
            pl.BlockSpec((1, _HH, _C), lambda b: (b, 1, 0)),
            pl.BlockSpec((1, _HW, 1), lambda b: (b, 0, 0)),
            pl.BlockSpec((1, 1, _P), lambda b: (b, 0, 0)),
            pl.BlockSpec((_C, _C), const),
            pl.BlockSpec((1, _C), const),
            pl.BlockSpec((_P, _P), const),
            pl.BlockSpec((1, _P), const),
            pl.BlockSpec((_C, 2 * _C + _P), const),
            pl.BlockSpec((1, _C), const),
            pl.BlockSpec((1, _C), const),
        ],
        out_specs=pl.BlockSpec((1, _HW, _C), lambda b: (b, 0, 0)),
        out_shape=jax.ShapeDtypeStruct((_B, _HW, _C), jnp.float32),
        compiler_params=pltpu.CompilerParams(
            dimension_semantics=("arbitrary",)),
    )(xt, xt, mask3d, gap3, w1, b1r, w3, b3r, blk, blkbr, c2r)


def kernel(x, proxy_IA_head, cl1_phi_w, cl1_phi_b, cl1_mlp_w, cl1_mlp_b,
           cl2_phi_w, cl2_phi_b, cl2_mlp_w, cl2_mlp_b,
           cl3_phi_w, cl3_phi_b, cl3_mlp_w, cl3_mlp_b,
           blk_mlp_w, blk_mlp_b):
    xt = x.reshape(_B, _C, _HW).transpose(0, 2, 1)
    proxyt = proxy_IA_head.reshape(_B, _P, _HW).transpose(0, 2, 1)

    p1, gap3 = _stage1(xt, proxyt,
                       cl1_phi_w.reshape(_C, 1), cl3_phi_w.reshape(_P, 1))

    mask1 = _sc_topk_mask()(p1.reshape(_B, _HW))

    out = _stage3(
        xt, mask1.reshape(_B, _HW, 1), gap3,
        cl1_mlp_w, cl1_mlp_b.reshape(1, _C),
        cl3_mlp_w, cl3_mlp_b.reshape(1, _P),
        blk_mlp_w, blk_mlp_b.reshape(1, _C), cl2_mlp_b.reshape(1, _C),
    )
    return out.transpose(0, 2, 1).reshape(_B, _C, _H, _W)


# speedup vs baseline: 1.1629x; 1.1629x over previous
import functools

import jax
import jax.numpy as jnp
from jax import lax
from jax.experimental import pallas as pl
from jax.experimental.pallas import tpu as pltpu
from jax.experimental.pallas import tpu_sc as plsc

_B, _C, _H, _W = 32, 768, 32, 32
_HW = _H * _W
_P = 400
_K1 = max(int(0.3 * _HW), 1)
_LANES = 16
_NSLICES = _HW // _LANES

_DN_RHS_T = (((1,), (1,)), ((), ()))


_NS = 4
_NSTEPS = _B // _NS


def _stage1_body(x_ref, proxy_ref, phi1_ref, phi3_ref, p1_ref, gap3_ref):
    phi1 = phi1_ref[...]
    phi3 = phi3_ref[...]
    for i in range(_NS):
        xb = x_ref[i]
        pb = proxy_ref[i]
        p1_ref[i] = jnp.dot(xb, phi1, preferred_element_type=jnp.float32)
        p3 = jnp.dot(pb, phi3, preferred_element_type=jnp.float32)
        mn = jnp.min(p3)
        m3 = (p3 > mn).astype(jnp.float32)
        gap3_ref[i] = jnp.sum(pb * m3, axis=0, keepdims=True) * (1.0 / _HW)


def _stage1(xt, proxyt, phi1_col, phi3_col):
    return pl.pallas_call(
        _stage1_body,
        grid=(_NSTEPS,),
        in_specs=[
            pl.BlockSpec((_NS, _HW, _C), lambda b: (b, 0, 0)),
            pl.BlockSpec((_NS, _HW, _P), lambda b: (b, 0, 0)),
            pl.BlockSpec((_C, 1), lambda b: (0, 0)),
            pl.BlockSpec((_P, 1), lambda b: (0, 0)),
        ],
        out_specs=[
            pl.BlockSpec((_NS, _HW, 1), lambda b: (b, 0, 0)),
            pl.BlockSpec((_NS, 1, _P), lambda b: (b, 0, 0)),
        ],
        out_shape=[
            jax.ShapeDtypeStruct((_B, _HW, 1), jnp.float32),
            jax.ShapeDtypeStruct((_B, 1, _P), jnp.float32),
        ],
        compiler_params=pltpu.CompilerParams(
            dimension_semantics=("arbitrary",),
            vmem_limit_bytes=120 * 1024 * 1024),
    )(xt, proxyt, phi1_col, phi3_col)


def _sc_body(p_hbm, mask_hbm, row_v, u_v, m_v):
    wid = lax.axis_index("s") * 2 + lax.axis_index("c")
    pltpu.sync_copy(p_hbm.at[wid], row_v)

    for j in range(_NSLICES):
        f = row_v[pl.ds(j * _LANES, _LANES)]
        u = lax.bitcast_convert_type(f, jnp.uint32)
        u = jnp.where(u >= jnp.uint32(0x80000000), ~u, u | jnp.uint32(0x80000000))
        u_v[pl.ds(j * _LANES, _LANES)] = u

    def _bit_step(s, t):
        bit = jnp.uint32(1) << (jnp.uint32(31) - s.astype(jnp.uint32))
        cand = t | bit
        acc = jnp.zeros((_LANES,), jnp.int32)
        for j in range(_NSLICES):
            u = u_v[pl.ds(j * _LANES, _LANES)]
            acc = acc + jnp.where(u >= cand, jnp.int32(1), jnp.int32(0))
        total = jnp.sum(acc)
        return jnp.where(total >= jnp.int32(_K1), cand, t)

    t = lax.fori_loop(0, 32, _bit_step, jnp.uint32(0))

    for j in range(_NSLICES):
        u = u_v[pl.ds(j * _LANES, _LANES)]
        m_v[pl.ds(j * _LANES, _LANES)] = jnp.where(u > t, 1.0, 0.0).astype(
            jnp.float32)
    pltpu.sync_copy(m_v, mask_hbm.at[wid])


@functools.cache
def _sc_topk_mask():
    return functools.partial(
        pl.kernel,
        out_type=jax.ShapeDtypeStruct((_B, _HW), jnp.float32),
        mesh=plsc.VectorSubcoreMesh(core_axis_name="c", subcore_axis_name="s"),
        compiler_params=pltpu.CompilerParams(needs_layout_passes=False),
        scratch_types=[
            pltpu.VMEM((_HW,), jnp.float32),
            pltpu.VMEM((_HW,), jnp.uint32),
            pltpu.VMEM((_HW,), jnp.float32),
        ],
    )(_sc_body)


def _stage3_body(x_ref, m1_ref, gap3_ref, w1_ref, b1_ref, w3_ref,
                 b3_ref, blk_ref, blkb_ref, c2_ref, out_ref):
    gaps = []
    for i in range(_NS):
        g = jnp.sum(x_ref[i] * m1_ref[i], axis=0, keepdims=True)
        gaps.append(g * (1.0 / _HW))
    gap1 = jnp.concatenate(gaps, axis=0)
    gap3 = gap3_ref[...].reshape(_NS, _P)
    cl1 = lax.dot_general(gap1, w1_ref[...], _DN_RHS_T,
                          preferred_element_type=jnp.float32) + b1_ref[...]
    cl3 = lax.dot_general(gap3, w3_ref[...], _DN_RHS_T,
                          preferred_element_type=jnp.float32) + b3_ref[...]
    c2 = jnp.broadcast_to(c2_ref[...], (_NS, _C))
    cat = jnp.concatenate([cl1, c2, cl3], axis=1)
    apre = lax.dot_general(cat, blk_ref[...], _DN_RHS_T,
                           preferred_element_type=jnp.float32) + blkb_ref[...]
    a = 1.0 + jnp.tanh(apre)
    for i in range(_NS):
        out_ref[i] = x_ref[i] * a[i:i + 1]


def _stage3(xt, mask3d, gap3, w1, b1r, w3, b3r, blk, blkbr, c2r):
    const = lambda b: (0, 0)
    return pl.pallas_call(
        _stage3_body,
        grid=(_NSTEPS,),
        in_specs=[
            pl.BlockSpec((_NS, _HW, _C), lambda b: (b, 0, 0)),
            pl.BlockSpec((_NS, _HW, 1), lambda b: (b, 0, 0)),
            pl.BlockSpec((_NS, 1, _P), lambda b: (b, 0, 0)),
            pl.BlockSpec((_C, _C), const),
            pl.BlockSpec((1, _C), const),
            pl.BlockSpec((_P, _P), const),
            pl.BlockSpec((1, _P), const),
            pl.BlockSpec((_C, 2 * _C + _P), const),
            pl.BlockSpec((1, _C), const),
            pl.BlockSpec((1, _C), const),
        ],
        out_specs=pl.BlockSpec((_NS, _HW, _C), lambda b: (b, 0, 0)),
        out_shape=jax.ShapeDtypeStruct((_B, _HW, _C), jnp.float32),
        compiler_params=pltpu.CompilerParams(
            dimension_semantics=("arbitrary",),
            vmem_limit_bytes=120 * 1024 * 1024),
    )(xt, mask3d, gap3, w1, b1r, w3, b3r, blk, blkbr, c2r)


def kernel(x, proxy_IA_head, cl1_phi_w, cl1_phi_b, cl1_mlp_w, cl1_mlp_b,
           cl2_phi_w, cl2_phi_b, cl2_mlp_w, cl2_mlp_b,
           cl3_phi_w, cl3_phi_b, cl3_mlp_w, cl3_mlp_b,
           blk_mlp_w, blk_mlp_b):
    xt = x.reshape(_B, _C, _HW).transpose(0, 2, 1)
    proxyt = proxy_IA_head.reshape(_B, _P, _HW).transpose(0, 2, 1)

    p1, gap3 = _stage1(xt, proxyt,
                       cl1_phi_w.reshape(_C, 1), cl3_phi_w.reshape(_P, 1))

    mask1 = _sc_topk_mask()(p1.reshape(_B, _HW))

    out = _stage3(
        xt, mask1.reshape(_B, _HW, 1), gap3,
        cl1_mlp_w, cl1_mlp_b.reshape(1, _C),
        cl3_mlp_w, cl3_mlp_b.reshape(1, _P),
        blk_mlp_w, blk_mlp_b.reshape(1, _C), cl2_mlp_b.reshape(1, _C),
    )
    return out.transpose(0, 2, 1).reshape(_B, _C, _H, _W)


# speedup vs baseline: 1.2593x; 1.0829x over previous
import functools

import jax
import jax.numpy as jnp
from jax import lax
from jax.experimental import pallas as pl
from jax.experimental.pallas import tpu as pltpu
from jax.experimental.pallas import tpu_sc as plsc

_B, _C, _H, _W = 32, 768, 32, 32
_HW = _H * _W
_P = 400
_K1 = max(int(0.3 * _HW), 1)
_LANES = 16
_NSLICES = _HW // _LANES

_DN_RHS_T = (((1,), (1,)), ((), ()))
_DN_STD = (((1,), (0,)), ((), ()))


_NS = 4
_NSTEPS = _B // _NS


def _stage1_body(x_ref, proxy_ref, phi1_ref, phi3_ref, p1_ref, gap3_ref):
    phi1 = phi1_ref[...]
    phi3 = phi3_ref[...]
    for i in range(_NS):
        xb = x_ref[i]
        pb = proxy_ref[i]
        p1_ref[i] = jnp.dot(xb, phi1, preferred_element_type=jnp.float32)
        p3 = jnp.dot(pb, phi3, preferred_element_type=jnp.float32)
        mn = jnp.min(p3)
        m3 = (p3 > mn).astype(jnp.float32)
        gap3_ref[i] = jnp.sum(pb * m3, axis=0, keepdims=True) * (1.0 / _HW)


def _stage1(xt, proxyt, phi1_col, phi3_col):
    return pl.pallas_call(
        _stage1_body,
        grid=(_NSTEPS,),
        in_specs=[
            pl.BlockSpec((_NS, _HW, _C), lambda b: (b, 0, 0)),
            pl.BlockSpec((_NS, _HW, _P), lambda b: (b, 0, 0)),
            pl.BlockSpec((_C, 1), lambda b: (0, 0)),
            pl.BlockSpec((_P, 1), lambda b: (0, 0)),
        ],
        out_specs=[
            pl.BlockSpec((_NS, _HW, 1), lambda b: (b, 0, 0)),
            pl.BlockSpec((_NS, 1, _P), lambda b: (b, 0, 0)),
        ],
        out_shape=[
            jax.ShapeDtypeStruct((_B, _HW, 1), jnp.float32),
            jax.ShapeDtypeStruct((_B, 1, _P), jnp.float32),
        ],
        compiler_params=pltpu.CompilerParams(
            dimension_semantics=("arbitrary",),
            vmem_limit_bytes=120 * 1024 * 1024),
    )(xt, proxyt, phi1_col, phi3_col)


def _sc_body(p_hbm, mask_hbm, row_v, u_v, m_v):
    wid = lax.axis_index("s") * 2 + lax.axis_index("c")
    pltpu.sync_copy(p_hbm.at[wid], row_v)

    for j in range(_NSLICES):
        f = row_v[pl.ds(j * _LANES, _LANES)]
        u = lax.bitcast_convert_type(f, jnp.uint32)
        u = jnp.where(u >= jnp.uint32(0x80000000), ~u, u | jnp.uint32(0x80000000))
        u_v[pl.ds(j * _LANES, _LANES)] = u

    def _bit_step(s, t):
        bit = jnp.uint32(1) << (jnp.uint32(31) - s.astype(jnp.uint32))
        cand = t | bit
        acc = jnp.zeros((_LANES,), jnp.int32)
        for j in range(_NSLICES):
            u = u_v[pl.ds(j * _LANES, _LANES)]
            acc = acc + jnp.where(u >= cand, jnp.int32(1), jnp.int32(0))
        total = jnp.sum(acc)
        return jnp.where(total >= jnp.int32(_K1), cand, t)

    t = lax.fori_loop(0, 32, _bit_step, jnp.uint32(0))

    for j in range(_NSLICES):
        u = u_v[pl.ds(j * _LANES, _LANES)]
        m_v[pl.ds(j * _LANES, _LANES)] = jnp.where(u > t, 1.0, 0.0).astype(
            jnp.float32)
    pltpu.sync_copy(m_v, mask_hbm.at[wid])


@functools.cache
def _sc_topk_mask():
    return functools.partial(
        pl.kernel,
        out_type=jax.ShapeDtypeStruct((_B, _HW), jnp.float32),
        mesh=plsc.VectorSubcoreMesh(core_axis_name="c", subcore_axis_name="s"),
        compiler_params=pltpu.CompilerParams(needs_layout_passes=False),
        scratch_types=[
            pltpu.VMEM((_HW,), jnp.float32),
            pltpu.VMEM((_HW,), jnp.uint32),
            pltpu.VMEM((_HW,), jnp.float32),
        ],
    )(_sc_body)


def _stage3_body(x_ref, m1_ref, gap3_ref, w1_ref, b1_ref, w3_ref,
                 b3_ref, blk_ref, blkb_ref, c2_ref, out_ref):
    gaps = []
    for i in range(_NS):
        g = lax.dot_general(m1_ref[0, i:i + 1], x_ref[i], _DN_STD,
                            preferred_element_type=jnp.float32)
        gaps.append(g * (1.0 / _HW))
    gap1 = jnp.concatenate(gaps, axis=0)
    gap3 = gap3_ref[...].reshape(_NS, _P)
    cl1 = lax.dot_general(gap1, w1_ref[...], _DN_RHS_T,
                          preferred_element_type=jnp.float32) + b1_ref[...]
    cl3 = lax.dot_general(gap3, w3_ref[...], _DN_RHS_T,
                          preferred_element_type=jnp.float32) + b3_ref[...]
    c2 = jnp.broadcast_to(c2_ref[...], (_NS, _C))
    cat = jnp.concatenate([cl1, c2, cl3], axis=1)
    apre = lax.dot_general(cat, blk_ref[...], _DN_RHS_T,
                           preferred_element_type=jnp.float32) + blkb_ref[...]
    a = 1.0 + jnp.tanh(apre)
    for i in range(_NS):
        out_ref[i] = x_ref[i] * a[i:i + 1]


def _stage3(xt, mask3d, gap3, w1, b1r, w3, b3r, blk, blkbr, c2r):
    const = lambda b: (0, 0)
    return pl.pallas_call(
        _stage3_body,
        grid=(_NSTEPS,),
        in_specs=[
            pl.BlockSpec((_NS, _HW, _C), lambda b: (b, 0, 0)),
            pl.BlockSpec((1, _NS, _HW), lambda b: (b, 0, 0)),
            pl.BlockSpec((_NS, 1, _P), lambda b: (b, 0, 0)),
            pl.BlockSpec((_C, _C), const),
            pl.BlockSpec((1, _C), const),
            pl.BlockSpec((_P, _P), const),
            pl.BlockSpec((1, _P), const),
            pl.BlockSpec((_C, 2 * _C + _P), const),
            pl.BlockSpec((1, _C), const),
            pl.BlockSpec((1, _C), const),
        ],
        out_specs=pl.BlockSpec((_NS, _HW, _C), lambda b: (b, 0, 0)),
        out_shape=jax.ShapeDtypeStruct((_B, _HW, _C), jnp.float32),
        compiler_params=pltpu.CompilerParams(
            dimension_semantics=("arbitrary",),
            vmem_limit_bytes=120 * 1024 * 1024),
    )(xt, mask3d, gap3, w1, b1r, w3, b3r, blk, blkbr, c2r)


def kernel(x, proxy_IA_head, cl1_phi_w, cl1_phi_b, cl1_mlp_w, cl1_mlp_b,
           cl2_phi_w, cl2_phi_b, cl2_mlp_w, cl2_mlp_b,
           cl3_phi_w, cl3_phi_b, cl3_mlp_w, cl3_mlp_b,
           blk_mlp_w, blk_mlp_b):
    xt = x.reshape(_B, _C, _HW).transpose(0, 2, 1)
    proxyt = proxy_IA_head.reshape(_B, _P, _HW).transpose(0, 2, 1)

    p1, gap3 = _stage1(xt, proxyt,
                       cl1_phi_w.reshape(_C, 1), cl3_phi_w.reshape(_P, 1))

    mask1 = _sc_topk_mask()(p1.reshape(_B, _HW))

    out = _stage3(
        xt, mask1.reshape(_NSTEPS, _NS, _HW), gap3,
        cl1_mlp_w, cl1_mlp_b.reshape(1, _C),
        cl3_mlp_w, cl3_mlp_b.reshape(1, _P),
        blk_mlp_w, blk_mlp_b.reshape(1, _C), cl2_mlp_b.reshape(1, _C),
    )
    return out.transpose(0, 2, 1).reshape(_B, _C, _H, _W)
